# baseline (speedup 1.0000x reference)
import jax
import jax.numpy as jnp
from jax import lax
from jax.experimental import pallas as pl
from jax.experimental.pallas import tpu as pltpu
from jax.experimental.pallas import tpu_sc as plsc

B = 4096
D = 64
L = 10
V = 100000
LANES = 16
NW = 32
RPW = B // NW
Q = 4
QB = B // Q
S = 8
SB = B // S
BLK = 1024
NBLK = B // BLK

def _sc_body(packed, ctx_t, tgt_idx, epacked, ctx_out, tgt_out, tgt2_out,
             nz_out, row_v, idx_v, acc_v, acc2_v, tidx_v, tt_v, erow_v, nz_v,
             sem, rsem):
    wid = lax.axis_index("s") * 2 + lax.axis_index("c")
    d0 = wid * 2
    d1 = wid * 2 + 1

    row_cp = pltpu.async_copy(packed.at[wid], row_v, rsem)

    pltpu.sync_copy(ctx_t.at[:, pl.ds(wid * RPW, RPW)], idx_v.at[:, pl.ds(0, RPW)])
    one = jnp.float32(1.0)
    zero = jnp.float32(0.0)
    for rc in range(RPW // LANES):
        cnt = jnp.zeros((LANES,), jnp.float32)
        for l in range(L):
            iv = idx_v[l, pl.ds(rc * LANES, LANES)]
            cnt = cnt + jnp.where(iv == 0, one, zero)
        nz_v[pl.ds(rc * LANES, LANES)] = cnt
    pltpu.sync_copy(nz_v, nz_out.at[pl.ds(wid * RPW, RPW)])

    pltpu.sync_copy(tgt_idx, tidx_v)

    def tt_chunk(ch, carry):
        for u in range(2):
            base = ch * 2 + u
            ii = tidx_v[pl.ds(base * LANES, LANES)]
            tt_v[pl.ds(base * LANES, LANES)] = plsc.load_gather(tidx_v, [ii])
        return carry

    lax.fori_loop(0, B // (2 * LANES), tt_chunk, 0)

    def unpack(pv):
        lo = plsc.bitcast(lax.shift_left(pv, jnp.int32(16)), jnp.float32)
        hi = plsc.bitcast(lax.bitwise_and(pv, jnp.int32(-65536)), jnp.float32)
        return lo, hi

    pltpu.sync_copy(epacked.at[wid], erow_v)
    for src, dst in ((tidx_v, tgt_out), (tt_v, tgt2_out)):
        for q in range(Q):
            def tgt_chunk_q(ch, carry, q=q, src=src):
                ii = src[pl.ds((q * (QB // LANES) + ch) * LANES, LANES)]
                lo, hi = unpack(plsc.load_gather(erow_v, [ii]))
                acc_v[pl.ds(ch * LANES, LANES)] = lo
                acc2_v[pl.ds(ch * LANES, LANES)] = hi
                return carry
            lax.fori_loop(0, QB // LANES, tgt_chunk_q, 0)
            pltpu.sync_copy(acc_v, dst.at[d0, pl.ds(q * QB, QB)])
            pltpu.sync_copy(acc2_v, dst.at[d1, pl.ds(q * QB, QB)])

    row_cp.wait()
    for s in range(S):
        pltpu.sync_copy(ctx_t.at[:, pl.ds(s * SB, SB)], idx_v.at[:, pl.ds(0, SB)])

        def ctx_chunk(ch, carry):
            a0 = jnp.zeros((LANES,), jnp.float32)
            a1 = jnp.zeros((LANES,), jnp.float32)
            for l in range(L):
                ii = idx_v[l, pl.ds(ch * LANES, LANES)]
                lo, hi = unpack(plsc.load_gather(row_v, [ii]))
                a0 = a0 + lo
                a1 = a1 + hi
            acc_v[pl.ds(ch * LANES, LANES)] = a0
            acc2_v[pl.ds(ch * LANES, LANES)] = a1
            return carry

        lax.fori_loop(0, SB // LANES, ctx_chunk, 0)
        pltpu.sync_copy(acc_v.at[pl.ds(0, SB)], ctx_out.at[d0, pl.ds(s * SB, SB)])
        pltpu.sync_copy(acc2_v.at[pl.ds(0, SB)], ctx_out.at[d1, pl.ds(s * SB, SB)])


_sc_gather = pl.kernel(
    _sc_body,
    out_type=(
        jax.ShapeDtypeStruct((D, B), jnp.float32),
        jax.ShapeDtypeStruct((D, B), jnp.float32),
        jax.ShapeDtypeStruct((D, B), jnp.float32),
        jax.ShapeDtypeStruct((B,), jnp.float32),
    ),
    mesh=plsc.VectorSubcoreMesh(core_axis_name="c", subcore_axis_name="s"),
    compiler_params=pltpu.CompilerParams(needs_layout_passes=False),
    scratch_types=[
        pltpu.VMEM((V,), jnp.int32),
        pltpu.VMEM((L, QB), jnp.int32),
        pltpu.VMEM((QB,), jnp.float32),
        pltpu.VMEM((QB,), jnp.float32),
        pltpu.VMEM((B,), jnp.int32),
        pltpu.VMEM((B,), jnp.int32),
        pltpu.VMEM((B,), jnp.int32),
        pltpu.VMEM((RPW,), jnp.float32),
        pltpu.SemaphoreType.DMA,
        pltpu.SemaphoreType.DMA,
    ],
)


def _tc_body(ctx_ref, tgt_ref, tgt2_ref, nz_ref, e0_ref, out_ref):
    j = pl.program_id(0)
    cb = ctx_ref[...] - e0_ref[...] * nz_ref[0, 0, :][None, :]
    ss = jnp.sum(cb * cb, axis=0, keepdims=True)
    cn = cb * lax.rsqrt(jnp.maximum(ss, 1e-24))
    logits = lax.dot_general(
        cn.astype(jnp.bfloat16), tgt_ref[...].astype(jnp.bfloat16),
        (((0,), (0,)), ((), ())),
        preferred_element_type=jnp.float32,
    )
    m = jnp.max(logits, axis=1, keepdims=True)
    lse = m[:, 0] + jnp.log(jnp.sum(jnp.exp(logits - m), axis=1))
    picked = jnp.sum(cn * tgt2_ref[...], axis=0)
    part = jnp.sum(lse - picked)

    @pl.when(j == 0)
    def _():
        out_ref[0, 0] = 0.0

    acc = out_ref[0, 0] + part
    out_ref[0, 0] = jnp.where(j == NBLK - 1, acc / B, acc)


def _pack_pairs(table_t):
    xb = table_t.astype(jnp.bfloat16).reshape(D // 2, 2, table_t.shape[1])
    u = lax.bitcast_convert_type(xb, jnp.uint16).astype(jnp.uint32)
    packed = (u[:, 1, :] << 16) | u[:, 0, :]
    return lax.bitcast_convert_type(packed, jnp.int32)


def kernel(contexts, targets, emb_in, emb_out):
    ctx_t = contexts.astype(jnp.int32).T
    tgt_i = targets.astype(jnp.int32)
    emb_t = emb_in.T
    packed = _pack_pairs(emb_t)
    epacked = _pack_pairs(emb_out.T[:, :B])
    ctx_raw_t, tgt_t, tgt2_t, nzero = _sc_gather(packed, ctx_t, tgt_i, epacked)
    nz3 = nzero.reshape(NBLK, 1, BLK)
    e0 = emb_t[:, 0:1].astype(jnp.bfloat16).astype(jnp.float32)
    loss = pl.pallas_call(
        _tc_body,
        grid=(NBLK,),
        in_specs=[
            pl.BlockSpec((D, BLK), lambda j: (0, j)),
            pl.BlockSpec((D, B), lambda j: (0, 0)),
            pl.BlockSpec((D, BLK), lambda j: (0, j)),
            pl.BlockSpec((1, 1, BLK), lambda j: (j, 0, 0)),
            pl.BlockSpec((D, 1), lambda j: (0, 0)),
        ],
        out_specs=pl.BlockSpec(memory_space=pltpu.SMEM),
        out_shape=jax.ShapeDtypeStruct((1, 1), jnp.float32),
    )(ctx_raw_t, tgt_t, tgt2_t, nz3, e0)
    return loss[0, 0]


# speedup vs baseline: 2.2603x; 2.2603x over previous
import jax
import jax.numpy as jnp
from jax import lax
from jax.experimental import pallas as pl
from jax.experimental.pallas import tpu as pltpu
from jax.experimental.pallas import tpu_sc as plsc

B = 4096
D = 64
L = 10
V = 100000
LANES = 16
NW = 32
DPW = D // NW
RPW = B // NW
Q = 4
QB = B // Q
BLK = 1024
NBLK = B // BLK


def _sc_body(emb_t, ctx_t, tgt_idx, eo_t, ctx_out, tgt_out, tgt2_out, nz_out,
             row_v, idx_v, acc_v, tidx_v, tt_v, erow_v, nz_v, sem, rsem):
    wid = lax.axis_index("s") * 2 + lax.axis_index("c")

    pltpu.sync_copy(ctx_t.at[:, pl.ds(wid * RPW, RPW)], idx_v.at[:, pl.ds(0, RPW)])
    one = jnp.float32(1.0)
    zero = jnp.float32(0.0)
    for rc in range(RPW // LANES):
        cnt = jnp.zeros((LANES,), jnp.float32)
        for l in range(L):
            iv = idx_v[l, pl.ds(rc * LANES, LANES)]
            cnt = cnt + jnp.where(iv == 0, one, zero)
        nz_v[pl.ds(rc * LANES, LANES)] = cnt
    pltpu.sync_copy(nz_v, nz_out.at[pl.ds(wid * RPW, RPW)])

    pltpu.sync_copy(tgt_idx, tidx_v)

    def tt_chunk(ch, carry):
        for u in range(2):
            base = ch * 2 + u
            ii = tidx_v[pl.ds(base * LANES, LANES)]
            tt_v[pl.ds(base * LANES, LANES)] = plsc.load_gather(tidx_v, [ii])
        return carry

    lax.fori_loop(0, B // (2 * LANES), tt_chunk, 0)

    for dd in range(DPW):
        d = wid * DPW + dd

        row_cp = pltpu.async_copy(emb_t.at[d], row_v, rsem)
        pltpu.sync_copy(eo_t.at[d, pl.ds(0, B)], erow_v)

        for src, dst in ((tidx_v, tgt_out), (tt_v, tgt2_out)):
            for q in range(Q):
                def tgt_chunk_q(ch, carry, q=q, src=src):
                    for u in range(2):
                        base = (q * (QB // LANES) + ch * 2 + u) * LANES
                        ii = src[pl.ds(base, LANES)]
                        acc_v[pl.ds((ch * 2 + u) * LANES, LANES)] = (
                            plsc.load_gather(erow_v, [ii]))
                    return carry
                lax.fori_loop(0, QB // (2 * LANES), tgt_chunk_q, 0)
                pltpu.sync_copy(acc_v, dst.at[d, pl.ds(q * QB, QB)])

        row_cp.wait()
        for q in range(Q):
            pltpu.sync_copy(ctx_t.at[:, pl.ds(q * QB, QB)], idx_v)

            def ctx_chunk(ch, carry):
                acc = jnp.zeros((LANES,), jnp.float32)
                for l in range(L):
                    ii = idx_v[l, pl.ds(ch * LANES, LANES)]
                    acc = acc + plsc.load_gather(row_v, [ii])
                acc_v[pl.ds(ch * LANES, LANES)] = acc
                return carry

            lax.fori_loop(0, QB // LANES, ctx_chunk, 0)
            pltpu.sync_copy(acc_v, ctx_out.at[d, pl.ds(q * QB, QB)])


_sc_gather = pl.kernel(
    _sc_body,
    out_type=(
        jax.ShapeDtypeStruct((D, B), jnp.float32),
        jax.ShapeDtypeStruct((D, B), jnp.float32),
        jax.ShapeDtypeStruct((D, B), jnp.float32),
        jax.ShapeDtypeStruct((B,), jnp.float32),
    ),
    mesh=plsc.VectorSubcoreMesh(core_axis_name="c", subcore_axis_name="s"),
    compiler_params=pltpu.CompilerParams(needs_layout_passes=False),
    scratch_types=[
        pltpu.VMEM((V,), jnp.float32),
        pltpu.VMEM((L, QB), jnp.int32),
        pltpu.VMEM((QB,), jnp.float32),
        pltpu.VMEM((B,), jnp.int32),
        pltpu.VMEM((B,), jnp.int32),
        pltpu.VMEM((B,), jnp.float32),
        pltpu.VMEM((RPW,), jnp.float32),
        pltpu.SemaphoreType.DMA,
        pltpu.SemaphoreType.DMA,
    ],
)


def _tc_body(ctx_ref, tgt_ref, tgt2_ref, nz_ref, e0_ref, out_ref):
    j = pl.program_id(0)
    cb = ctx_ref[...] - e0_ref[...] * nz_ref[0, 0, :][None, :]
    ss = jnp.sum(cb * cb, axis=0, keepdims=True)
    cn = cb * lax.rsqrt(jnp.maximum(ss, 1e-24))
    logits = lax.dot_general(
        cn.astype(jnp.bfloat16), tgt_ref[...].astype(jnp.bfloat16),
        (((0,), (0,)), ((), ())),
        preferred_element_type=jnp.float32,
    )
    m = jnp.max(logits, axis=1, keepdims=True)
    lse = m[:, 0] + jnp.log(jnp.sum(jnp.exp(logits - m), axis=1))
    picked = jnp.sum(cn * tgt2_ref[...], axis=0)
    part = jnp.sum(lse - picked)

    @pl.when(j == 0)
    def _():
        out_ref[0, 0] = 0.0

    acc = out_ref[0, 0] + part
    out_ref[0, 0] = jnp.where(j == NBLK - 1, acc / B, acc)


def kernel(contexts, targets, emb_in, emb_out):
    ctx_t = contexts.astype(jnp.int32).T
    tgt_i = targets.astype(jnp.int32)
    emb_t = emb_in.T
    eo_t = emb_out.T
    ctx_raw_t, tgt_t, tgt2_t, nzero = _sc_gather(emb_t, ctx_t, tgt_i, eo_t)
    nz3 = nzero.reshape(NBLK, 1, BLK)
    e0 = emb_t[:, 0:1]
    loss = pl.pallas_call(
        _tc_body,
        grid=(NBLK,),
        in_specs=[
            pl.BlockSpec((D, BLK), lambda j: (0, j)),
            pl.BlockSpec((D, B), lambda j: (0, 0)),
            pl.BlockSpec((D, BLK), lambda j: (0, j)),
            pl.BlockSpec((1, 1, BLK), lambda j: (j, 0, 0)),
            pl.BlockSpec((D, 1), lambda j: (0, 0)),
        ],
        out_specs=pl.BlockSpec(memory_space=pltpu.SMEM),
        out_shape=jax.ShapeDtypeStruct((1, 1), jnp.float32),
    )(ctx_raw_t, tgt_t, tgt2_t, nz3, e0)
    return loss[0, 0]


# speedup vs baseline: 2.3465x; 1.0381x over previous
import jax
import jax.numpy as jnp
from jax import lax
from jax.experimental import pallas as pl
from jax.experimental.pallas import tpu as pltpu
from jax.experimental.pallas import tpu_sc as plsc

B = 4096
D = 64
L = 10
V = 100000
LANES = 16
NW = 32
DPW = D // NW
RPW = B // NW
Q = 4
QB = B // Q
S = 8
SB = B // S
BLK = 2048
NBLK = B // BLK


def _sc_body(emb_t, ctx_t, tgt_idx, eo_t, ctx_out, tgt_out, tgt2_out, nz_out,
             row_v, idx_v, idx2_v, acc_v, tidx_v, tt_v, erow_v, nz_v, sem, rsem):
    wid = lax.axis_index("s") * 2 + lax.axis_index("c")

    pltpu.sync_copy(ctx_t.at[:, pl.ds(wid * RPW, RPW)], idx_v.at[:, pl.ds(0, RPW)])
    one = jnp.float32(1.0)
    zero = jnp.float32(0.0)
    for rc in range(RPW // LANES):
        cnt = jnp.zeros((LANES,), jnp.float32)
        for l in range(L):
            iv = idx_v[l, pl.ds(rc * LANES, LANES)]
            cnt = cnt + jnp.where(iv == 0, one, zero)
        nz_v[pl.ds(rc * LANES, LANES)] = cnt
    pltpu.sync_copy(nz_v, nz_out.at[pl.ds(wid * RPW, RPW)])

    pltpu.sync_copy(tgt_idx, tidx_v)

    def tt_chunk(ch, carry):
        for u in range(2):
            base = ch * 2 + u
            ii = tidx_v[pl.ds(base * LANES, LANES)]
            tt_v[pl.ds(base * LANES, LANES)] = plsc.load_gather(tidx_v, [ii])
        return carry

    lax.fori_loop(0, B // (2 * LANES), tt_chunk, 0)

    for dd in range(DPW):
        d = wid * DPW + dd

        row_cp = pltpu.async_copy(emb_t.at[d], row_v, rsem)
        pltpu.sync_copy(eo_t.at[d, pl.ds(0, B)], erow_v)

        for src, dst in ((tidx_v, tgt_out), (tt_v, tgt2_out)):
            for q in range(Q):
                def tgt_chunk_q(ch, carry, q=q, src=src):
                    for u in range(4):
                        base = (q * (QB // LANES) + ch * 4 + u) * LANES
                        ii = src[pl.ds(base, LANES)]
                        acc_v[pl.ds((ch * 4 + u) * LANES, LANES)] = (
                            plsc.load_gather(erow_v, [ii]))
                    return carry
                lax.fori_loop(0, QB // (4 * LANES), tgt_chunk_q, 0)
                pltpu.sync_copy(acc_v, dst.at[d, pl.ds(q * QB, QB)])

        idx_cps = [pltpu.async_copy(ctx_t.at[:, pl.ds(0, SB)], idx_v, sem)]
        row_cp.wait()
        for s in range(S):
            idx_cps[s].wait()
            if s + 1 < S:
                nxt = idx2_v if s % 2 == 0 else idx_v
                idx_cps.append(pltpu.async_copy(
                    ctx_t.at[:, pl.ds((s + 1) * SB, SB)], nxt, sem))
            cur = idx_v if s % 2 == 0 else idx2_v

            def ctx_chunk(ch, carry, cur=cur):
                acc = jnp.zeros((LANES,), jnp.float32)
                for l in range(L):
                    ii = cur[l, pl.ds(ch * LANES, LANES)]
                    acc = acc + plsc.load_gather(row_v, [ii])
                acc_v[pl.ds(ch * LANES, LANES)] = acc
                return carry

            lax.fori_loop(0, SB // LANES, ctx_chunk, 0)
            pltpu.sync_copy(acc_v.at[pl.ds(0, SB)], ctx_out.at[d, pl.ds(s * SB, SB)])


_sc_gather = pl.kernel(
    _sc_body,
    out_type=(
        jax.ShapeDtypeStruct((D, B), jnp.float32),
        jax.ShapeDtypeStruct((D, B), jnp.float32),
        jax.ShapeDtypeStruct((D, B), jnp.float32),
        jax.ShapeDtypeStruct((B,), jnp.float32),
    ),
    mesh=plsc.VectorSubcoreMesh(core_axis_name="c", subcore_axis_name="s"),
    compiler_params=pltpu.CompilerParams(needs_layout_passes=False),
    scratch_types=[
        pltpu.VMEM((V,), jnp.float32),
        pltpu.VMEM((L, SB), jnp.int32),
        pltpu.VMEM((L, SB), jnp.int32),
        pltpu.VMEM((QB,), jnp.float32),
        pltpu.VMEM((B,), jnp.int32),
        pltpu.VMEM((B,), jnp.int32),
        pltpu.VMEM((B,), jnp.float32),
        pltpu.VMEM((RPW,), jnp.float32),
        pltpu.SemaphoreType.DMA,
        pltpu.SemaphoreType.DMA,
    ],
)


def _tc_body(ctx_ref, tgt_ref, tgt2_ref, nz_ref, e0_ref, out_ref):
    j = pl.program_id(0)
    cb = ctx_ref[...] - e0_ref[...] * nz_ref[0, 0, :][None, :]
    ss = jnp.sum(cb * cb, axis=0, keepdims=True)
    cn = cb * lax.rsqrt(jnp.maximum(ss, 1e-24))
    logits = lax.dot_general(
        cn.astype(jnp.bfloat16), tgt_ref[...].astype(jnp.bfloat16),
        (((0,), (0,)), ((), ())),
        preferred_element_type=jnp.float32,
    )
    m = jnp.max(logits, axis=1, keepdims=True)
    lse = m[:, 0] + jnp.log(jnp.sum(jnp.exp(logits - m), axis=1))
    picked = jnp.sum(cn * tgt2_ref[...], axis=0)
    part = jnp.sum(lse - picked)

    @pl.when(j == 0)
    def _():
        out_ref[0, 0] = 0.0

    acc = out_ref[0, 0] + part
    out_ref[0, 0] = jnp.where(j == NBLK - 1, acc / B, acc)


def kernel(contexts, targets, emb_in, emb_out):
    ctx_t = contexts.astype(jnp.int32).T
    tgt_i = targets.astype(jnp.int32)
    emb_t = emb_in.T
    eo_t = emb_out.T
    ctx_raw_t, tgt_t, tgt2_t, nzero = _sc_gather(emb_t, ctx_t, tgt_i, eo_t)
    nz3 = nzero.reshape(NBLK, 1, BLK)
    e0 = emb_t[:, 0:1]
    loss = pl.pallas_call(
        _tc_body,
        grid=(NBLK,),
        in_specs=[
            pl.BlockSpec((D, BLK), lambda j: (0, j)),
            pl.BlockSpec((D, B), lambda j: (0, 0)),
            pl.BlockSpec((D, BLK), lambda j: (0, j)),
            pl.BlockSpec((1, 1, BLK), lambda j: (j, 0, 0)),
            pl.BlockSpec((D, 1), lambda j: (0, 0)),
        ],
        out_specs=pl.BlockSpec(memory_space=pltpu.SMEM),
        out_shape=jax.ShapeDtypeStruct((1, 1), jnp.float32),
    )(ctx_raw_t, tgt_t, tgt2_t, nz3, e0)
    return loss[0, 0]
